# baseline (speedup 1.0000x reference)
import jax
import jax.numpy as jnp
from jax import lax
from jax.experimental import pallas as pl
from jax.experimental.pallas import tpu as pltpu
from jax.experimental.pallas import tpu_sc as plsc

N = 50000
IN_F = 128
OUT_F = 128
K = 7

NC = 2
NS = 16
NW = NC * NS
L = 16

G = 48
CH = 34
RPW = G * CH
GR = K * G
NV = GR // L
Q = 3
GQ = GR // Q
RU = 4

TB = 5000


def _tc_matmul_body(x_ref, wr_ref, y_ref):
    xb = x_ref[...]
    for k in range(K):
        y_ref[k] = lax.dot_general(
            xb, wr_ref[k],
            (((1,), (0,)), ((), ())),
            preferred_element_type=jnp.float32,
        )


def _tc_matmul(x, wr3):
    return pl.pallas_call(
        _tc_matmul_body,
        grid=(N // TB,),
        in_specs=[
            pl.BlockSpec((TB, IN_F), lambda i: (i, 0)),
            pl.BlockSpec((K, IN_F, OUT_F), lambda i: (0, 0, 0)),
        ],
        out_specs=pl.BlockSpec((K, TB, OUT_F), lambda i: (0, i, 0)),
        out_shape=jax.ShapeDtypeStruct((K, N, OUT_F), jnp.float32),
    )(x, wr3)


def _sc_body(y_hbm, nb_hbm, b_hbm, out_hbm, idxbuf, fidxbuf, gbuf, obuf, bbuf, sem):
    c = lax.axis_index("c")
    s = lax.axis_index("s")
    wid = s * NC + c
    base_row = lax.min(wid * RPW, N - RPW)

    pltpu.sync_copy(nb_hbm.at[pl.ds(base_row * K, K * RPW)], idxbuf)
    pltpu.sync_copy(b_hbm, bbuf)
    bvecs = [bbuf[pl.ds(cc * L, L)] for cc in range(OUT_F // L)]

    def fire(ch, par):
        for v in range(NV):
            lane = lax.iota(jnp.int32, L) + (v * L)
            pat = lax.rem(lane, K)
            nbv = idxbuf[pl.ds(ch * GR + v * L, L)]
            fidxbuf[par, v // K, pl.ds((v % K) * L, L)] = nbv + pat * N
        for q in range(Q):
            pltpu.make_async_copy(
                y_hbm.at[fidxbuf.at[par, q]],
                gbuf.at[pl.ds(par * GR + q * GQ, GQ)],
                sem,
            ).start()

    def drain(par):
        for q in range(Q):
            pltpu.make_async_copy(
                y_hbm.at[fidxbuf.at[par, q]],
                gbuf.at[pl.ds(par * GR + q * GQ, GQ)],
                sem,
            ).wait()

    def sum_chunk(ch, par):
        gb = par * GR

        def row_body(rr, carry2):
            for u in range(RU):
                g = gb + (rr * RU + u) * K
                r = rr * RU + u
                for cc in range(OUT_F // L):
                    v = [gbuf[g + t, pl.ds(cc * L, L)] for t in range(K)]
                    s01 = v[0] + v[1]
                    s23 = v[2] + v[3]
                    s45 = v[4] + v[5]
                    s6b = v[6] + bvecs[cc]
                    obuf[r, pl.ds(cc * L, L)] = (s01 + s23) + (s45 + s6b)
            return carry2

        lax.fori_loop(0, G // RU, row_body, 0)
        pltpu.sync_copy(obuf, out_hbm.at[pl.ds(base_row + ch * G, G)])

    fire(0, 0)

    def pair_body(h, carry):
        ch0 = 2 * h
        ch1 = ch0 + 1
        fire(ch1, 1)
        drain(0)
        sum_chunk(ch0, 0)

        @pl.when(ch1 + 1 < CH)
        def _():
            fire(ch1 + 1, 0)

        drain(1)
        sum_chunk(ch1, 1)
        return carry

    lax.fori_loop(0, CH // 2, pair_body, 0)


def _sc_gather_sum(yf, nb_pad, b):
    mesh = plsc.VectorSubcoreMesh(
        core_axis_name="c", subcore_axis_name="s", num_cores=NC, num_subcores=NS
    )
    return pl.kernel(
        _sc_body,
        out_type=jax.ShapeDtypeStruct((N, OUT_F), jnp.float32),
        mesh=mesh,
        scratch_types=[
            pltpu.VMEM((K * RPW,), jnp.int32),
            pltpu.VMEM((2, Q, GQ), jnp.int32),
            pltpu.VMEM((2 * GR, OUT_F), jnp.float32),
            pltpu.VMEM((G, OUT_F), jnp.float32),
            pltpu.VMEM((OUT_F,), jnp.float32),
            pltpu.SemaphoreType.DMA,
        ],
    )(yf, nb_pad, b)


def kernel(x, neigh_orders, W, b):
    wr3 = W.reshape(OUT_F, K, IN_F).transpose(1, 2, 0)
    y3 = _tc_matmul(x, wr3)
    yf = y3.reshape(K * N, OUT_F)
    nb = neigh_orders.astype(jnp.int32)
    return _sc_gather_sum(yf, nb, b)


# speedup vs baseline: 1.0039x; 1.0039x over previous
import jax
import jax.numpy as jnp
from jax import lax
from jax.experimental import pallas as pl
from jax.experimental.pallas import tpu as pltpu
from jax.experimental.pallas import tpu_sc as plsc

N = 50000
IN_F = 128
OUT_F = 128
K = 7

NC = 2
NS = 16
NW = NC * NS
L = 16

G = 48
CH = 34
RPW = G * CH
GR = K * G
NV = GR // L
Q = 3
GQ = GR // Q
RU = 4

TB = 2000


def _tc_matmul_body(x_ref, wr_ref, y_ref):
    xb = x_ref[...]
    for k in range(K):
        y_ref[k] = lax.dot_general(
            xb, wr_ref[k],
            (((1,), (0,)), ((), ())),
            preferred_element_type=jnp.float32,
        )


def _tc_matmul(x, wr3):
    return pl.pallas_call(
        _tc_matmul_body,
        grid=(N // TB,),
        in_specs=[
            pl.BlockSpec((TB, IN_F), lambda i: (i, 0)),
            pl.BlockSpec((K, IN_F, OUT_F), lambda i: (0, 0, 0)),
        ],
        out_specs=pl.BlockSpec((K, TB, OUT_F), lambda i: (0, i, 0)),
        out_shape=jax.ShapeDtypeStruct((K, N, OUT_F), jnp.float32),
    )(x, wr3)


def _sc_body(y_hbm, nb_hbm, b_hbm, out_hbm, idxbuf, fidxbuf, gbuf, obuf, bbuf, sem):
    c = lax.axis_index("c")
    s = lax.axis_index("s")
    wid = s * NC + c
    base_row = lax.min(wid * RPW, N - RPW)

    pltpu.sync_copy(nb_hbm.at[pl.ds(base_row * K, K * RPW)], idxbuf)
    pltpu.sync_copy(b_hbm, bbuf)
    bvecs = [bbuf[pl.ds(cc * L, L)] for cc in range(OUT_F // L)]

    def fire(ch, par):
        for v in range(NV):
            lane = lax.iota(jnp.int32, L) + (v * L)
            pat = lax.rem(lane, K)
            nbv = idxbuf[pl.ds(ch * GR + v * L, L)]
            fidxbuf[par, v // K, pl.ds((v % K) * L, L)] = nbv + pat * N
        for q in range(Q):
            pltpu.make_async_copy(
                y_hbm.at[fidxbuf.at[par, q]],
                gbuf.at[pl.ds(par * GR + q * GQ, GQ)],
                sem,
            ).start()

    def drain(par):
        for q in range(Q):
            pltpu.make_async_copy(
                y_hbm.at[fidxbuf.at[par, q]],
                gbuf.at[pl.ds(par * GR + q * GQ, GQ)],
                sem,
            ).wait()

    def sum_chunk(ch, par):
        gb = par * GR

        def row_body(rr, carry2):
            for u in range(RU):
                g = gb + (rr * RU + u) * K
                r = rr * RU + u
                for cc in range(OUT_F // L):
                    v = [gbuf[g + t, pl.ds(cc * L, L)] for t in range(K)]
                    s01 = v[0] + v[1]
                    s23 = v[2] + v[3]
                    s45 = v[4] + v[5]
                    s6b = v[6] + bvecs[cc]
                    obuf[r, pl.ds(cc * L, L)] = (s01 + s23) + (s45 + s6b)
            return carry2

        lax.fori_loop(0, G // RU, row_body, 0)
        pltpu.sync_copy(obuf, out_hbm.at[pl.ds(base_row + ch * G, G)])

    fire(0, 0)

    def pair_body(h, carry):
        ch0 = 2 * h
        ch1 = ch0 + 1
        fire(ch1, 1)
        drain(0)
        sum_chunk(ch0, 0)

        @pl.when(ch1 + 1 < CH)
        def _():
            fire(ch1 + 1, 0)

        drain(1)
        sum_chunk(ch1, 1)
        return carry

    lax.fori_loop(0, CH // 2, pair_body, 0)


def _sc_gather_sum(yf, nb_pad, b):
    mesh = plsc.VectorSubcoreMesh(
        core_axis_name="c", subcore_axis_name="s", num_cores=NC, num_subcores=NS
    )
    return pl.kernel(
        _sc_body,
        out_type=jax.ShapeDtypeStruct((N, OUT_F), jnp.float32),
        mesh=mesh,
        scratch_types=[
            pltpu.VMEM((K * RPW,), jnp.int32),
            pltpu.VMEM((2, Q, GQ), jnp.int32),
            pltpu.VMEM((2 * GR, OUT_F), jnp.float32),
            pltpu.VMEM((G, OUT_F), jnp.float32),
            pltpu.VMEM((OUT_F,), jnp.float32),
            pltpu.SemaphoreType.DMA,
        ],
    )(yf, nb_pad, b)


def kernel(x, neigh_orders, W, b):
    wr3 = W.reshape(OUT_F, K, IN_F).transpose(1, 2, 0)
    y3 = _tc_matmul(x, wr3)
    yf = y3.reshape(K * N, OUT_F)
    nb = neigh_orders.astype(jnp.int32)
    return _sc_gather_sum(yf, nb, b)


# speedup vs baseline: 1.0144x; 1.0104x over previous
import jax
import jax.numpy as jnp
from jax import lax
from jax.experimental import pallas as pl
from jax.experimental.pallas import tpu as pltpu
from jax.experimental.pallas import tpu_sc as plsc

N = 50000
IN_F = 128
OUT_F = 128
K = 7

NC = 2
NS = 16
NW = NC * NS
L = 16

G = 32
CH = 50
RPW = G * CH
GR = K * G
NV = GR // L
Q = 2
GQ = GR // Q
RU = 4

TB = 2000


def _tc_matmul_body(x_ref, wr_ref, y_ref):
    xb = x_ref[...]
    for k in range(K):
        y_ref[k] = lax.dot_general(
            xb, wr_ref[k],
            (((1,), (0,)), ((), ())),
            preferred_element_type=jnp.float32,
        )


def _tc_matmul(x, wr3):
    return pl.pallas_call(
        _tc_matmul_body,
        grid=(N // TB,),
        in_specs=[
            pl.BlockSpec((TB, IN_F), lambda i: (i, 0)),
            pl.BlockSpec((K, IN_F, OUT_F), lambda i: (0, 0, 0)),
        ],
        out_specs=pl.BlockSpec((K, TB, OUT_F), lambda i: (0, i, 0)),
        out_shape=jax.ShapeDtypeStruct((K, N, OUT_F), jnp.float32),
    )(x, wr3)


def _sc_body(y_hbm, nb_hbm, b_hbm, out_hbm, idxbuf, fidxbuf, gbuf, obuf, bbuf, sem):
    c = lax.axis_index("c")
    s = lax.axis_index("s")
    wid = s * NC + c
    base_row = lax.min(wid * RPW, N - RPW)

    pltpu.sync_copy(nb_hbm.at[pl.ds(base_row * K, K * RPW)], idxbuf)
    pltpu.sync_copy(b_hbm, bbuf)
    bvecs = [bbuf[pl.ds(cc * L, L)] for cc in range(OUT_F // L)]

    def fire(ch, par):
        for v in range(NV):
            lane = lax.iota(jnp.int32, L) + (v * L)
            pat = lax.rem(lane, K)
            nbv = idxbuf[pl.ds(ch * GR + v * L, L)]
            fidxbuf[par, v // K, pl.ds((v % K) * L, L)] = nbv + pat * N
        for q in range(Q):
            pltpu.make_async_copy(
                y_hbm.at[fidxbuf.at[par, q]],
                gbuf.at[pl.ds(par * GR + q * GQ, GQ)],
                sem,
            ).start()

    def drain(par):
        for q in range(Q):
            pltpu.make_async_copy(
                y_hbm.at[fidxbuf.at[par, q]],
                gbuf.at[pl.ds(par * GR + q * GQ, GQ)],
                sem,
            ).wait()

    def sum_chunk(ch, par):
        gb = par * GR

        def row_body(rr, carry2):
            for u in range(RU):
                g = gb + (rr * RU + u) * K
                r = rr * RU + u
                for cc in range(OUT_F // L):
                    v = [gbuf[g + t, pl.ds(cc * L, L)] for t in range(K)]
                    s01 = v[0] + v[1]
                    s23 = v[2] + v[3]
                    s45 = v[4] + v[5]
                    s6b = v[6] + bvecs[cc]
                    obuf[r, pl.ds(cc * L, L)] = (s01 + s23) + (s45 + s6b)
            return carry2

        lax.fori_loop(0, G // RU, row_body, 0)
        pltpu.sync_copy(obuf, out_hbm.at[pl.ds(base_row + ch * G, G)])

    fire(0, 0)

    def pair_body(h, carry):
        ch0 = 2 * h
        ch1 = ch0 + 1
        fire(ch1, 1)
        drain(0)
        sum_chunk(ch0, 0)

        @pl.when(ch1 + 1 < CH)
        def _():
            fire(ch1 + 1, 0)

        drain(1)
        sum_chunk(ch1, 1)
        return carry

    lax.fori_loop(0, CH // 2, pair_body, 0)


def _sc_gather_sum(yf, nb_pad, b):
    mesh = plsc.VectorSubcoreMesh(
        core_axis_name="c", subcore_axis_name="s", num_cores=NC, num_subcores=NS
    )
    return pl.kernel(
        _sc_body,
        out_type=jax.ShapeDtypeStruct((N, OUT_F), jnp.float32),
        mesh=mesh,
        scratch_types=[
            pltpu.VMEM((K * RPW,), jnp.int32),
            pltpu.VMEM((2, Q, GQ), jnp.int32),
            pltpu.VMEM((2 * GR, OUT_F), jnp.float32),
            pltpu.VMEM((G, OUT_F), jnp.float32),
            pltpu.VMEM((OUT_F,), jnp.float32),
            pltpu.SemaphoreType.DMA,
        ],
    )(yf, nb_pad, b)


def kernel(x, neigh_orders, W, b):
    wr3 = W.reshape(OUT_F, K, IN_F).transpose(1, 2, 0)
    y3 = _tc_matmul(x, wr3)
    yf = y3.reshape(K * N, OUT_F)
    nb = neigh_orders.astype(jnp.int32)
    return _sc_gather_sum(yf, nb, b)
